# speedup vs baseline: 4.0943x; 4.0943x over previous
import jax
import jax.numpy as jnp
from jax.experimental import pallas as pl
from jax.experimental.pallas import tpu as pltpu

_BATCH = 4096
_HIST = 50
_DIM = 64
_ROWS = _BATCH * _HIST
_BLK = 8192


def _select_kernel(x_ref, rows_ref, out_ref):
    mask = x_ref[...] == 1
    t0 = rows_ref[0, :]
    t1 = rows_ref[1, :]
    out_ref[...] = jnp.where(mask, t1[None, :], t0[None, :])


def kernel(x, table):
    x2d = x.astype(jnp.int32).reshape(_ROWS, 1)
    rows01 = table[:2]
    out2d = pl.pallas_call(
        _select_kernel,
        grid=(_ROWS // _BLK,),
        in_specs=[
            pl.BlockSpec((_BLK, 1), lambda i: (i, 0)),
            pl.BlockSpec((2, _DIM), lambda i: (0, 0)),
        ],
        out_specs=pl.BlockSpec((_BLK, _DIM), lambda i: (i, 0)),
        out_shape=jax.ShapeDtypeStruct((_ROWS, _DIM), table.dtype),
        compiler_params=pltpu.CompilerParams(
            dimension_semantics=("parallel",),
        ),
    )(x2d, rows01)
    return out2d.reshape(_BATCH, _HIST, _DIM)


# speedup vs baseline: 5.2151x; 1.2737x over previous
import jax
import jax.numpy as jnp
from jax.experimental import pallas as pl
from jax.experimental.pallas import tpu as pltpu

_BATCH = 4096
_HIST = 50
_DIM = 64
_ROWS = _BATCH * _HIST
_MCOL = 128
_MROW = _ROWS // _MCOL
_OUTC = _MCOL * _DIM
_BLKR = 64


def _select_kernel(x_ref, rows_ref, out_ref):
    xrep = jnp.repeat(x_ref[...], _DIM, axis=1)
    out_ref[...] = jnp.where(xrep == 1, rows_ref[1, :], rows_ref[0, :])


def kernel(x, table):
    xm = x.astype(jnp.int32).reshape(_MROW, _MCOL)
    rows01 = jnp.tile(table[:2], (1, _MCOL))
    out2d = pl.pallas_call(
        _select_kernel,
        grid=(_MROW // _BLKR,),
        in_specs=[
            pl.BlockSpec((_BLKR, _MCOL), lambda i: (i, 0)),
            pl.BlockSpec((2, _OUTC), lambda i: (0, 0)),
        ],
        out_specs=pl.BlockSpec((_BLKR, _OUTC), lambda i: (i, 0)),
        out_shape=jax.ShapeDtypeStruct((_MROW, _OUTC), table.dtype),
        compiler_params=pltpu.CompilerParams(
            dimension_semantics=("parallel",),
        ),
    )(xm, rows01)
    return out2d.reshape(_BATCH, _HIST, _DIM)


# speedup vs baseline: 9.2530x; 1.7743x over previous
import jax
import jax.numpy as jnp
from jax.experimental import pallas as pl
from jax.experimental.pallas import tpu as pltpu

_BATCH = 4096
_HIST = 50
_DIM = 64
_BB = 256


def _select_kernel(x_ref, rows_ref, out_ref):
    x3 = jnp.broadcast_to(x_ref[...][:, :, None], (_BB, _HIST, _DIM))
    out_ref[...] = jnp.where(x3 == 1, rows_ref[1, :], rows_ref[0, :])


def kernel(x, table):
    xi = x.astype(jnp.int32)
    rows01 = table[:2]
    out = pl.pallas_call(
        _select_kernel,
        grid=(_BATCH // _BB,),
        in_specs=[
            pl.BlockSpec((_BB, _HIST), lambda i: (i, 0)),
            pl.BlockSpec((2, _DIM), lambda i: (0, 0)),
        ],
        out_specs=pl.BlockSpec((_BB, _HIST, _DIM), lambda i: (i, 0, 0)),
        out_shape=jax.ShapeDtypeStruct((_BATCH, _HIST, _DIM), table.dtype),
        compiler_params=pltpu.CompilerParams(
            dimension_semantics=("parallel",),
        ),
    )(xi, rows01)
    return out


# speedup vs baseline: 9.2857x; 1.0035x over previous
import jax
import jax.numpy as jnp
from jax.experimental import pallas as pl
from jax.experimental.pallas import tpu as pltpu

_BATCH = 4096
_HIST = 50
_DIM = 64
_BB = 512


def _select_kernel(x_ref, rows_ref, out_ref):
    x3 = jnp.broadcast_to(x_ref[...][:, :, None], (_BB, _HIST, _DIM))
    out_ref[...] = jnp.where(x3 == 1, rows_ref[1, :], rows_ref[0, :])


def kernel(x, table):
    xi = x.astype(jnp.int32)
    rows01 = table[:2]
    out = pl.pallas_call(
        _select_kernel,
        grid=(_BATCH // _BB,),
        in_specs=[
            pl.BlockSpec((_BB, _HIST), lambda i: (i, 0)),
            pl.BlockSpec((2, _DIM), lambda i: (0, 0)),
        ],
        out_specs=pl.BlockSpec((_BB, _HIST, _DIM), lambda i: (i, 0, 0)),
        out_shape=jax.ShapeDtypeStruct((_BATCH, _HIST, _DIM), table.dtype),
        compiler_params=pltpu.CompilerParams(
            dimension_semantics=("parallel",),
        ),
    )(xi, rows01)
    return out


# speedup vs baseline: 9.7738x; 1.0526x over previous
import jax
import jax.numpy as jnp
from jax.experimental import pallas as pl
from jax.experimental.pallas import tpu as pltpu

_BATCH = 4096
_HIST = 50
_DIM = 64
_BB = 256
_GRID = _BATCH // _BB
_NBUF = 4


def _copy(scratch_ref, out_ref, sem_ref, slot, i):
    return pltpu.make_async_copy(
        scratch_ref.at[slot],
        out_ref.at[pl.ds(i * _BB, _BB)],
        sem_ref.at[slot],
    )


def _select_kernel(x_ref, rows_ref, out_ref, scratch_ref, sem_ref):
    i = pl.program_id(0)
    slot = jax.lax.rem(i, _NBUF)

    @pl.when(i >= _NBUF)
    def _wait_prev():
        _copy(scratch_ref, out_ref, sem_ref, slot, i - _NBUF).wait()

    x3 = jnp.broadcast_to(x_ref[...][:, :, None], (_BB, _HIST, _DIM))
    scratch_ref[slot] = jnp.where(x3 == 1, rows_ref[1, :], rows_ref[0, :])
    _copy(scratch_ref, out_ref, sem_ref, slot, i).start()

    @pl.when(i == _GRID - 1)
    def _drain_all():
        for k in range(_NBUF):
            s = jax.lax.rem(i + 1 + k, _NBUF)
            _copy(scratch_ref, out_ref, sem_ref, s, i - (_NBUF - 1 - k)).wait()


def kernel(x, table):
    xi = x.astype(jnp.int32)
    rows01 = table[:2]
    out = pl.pallas_call(
        _select_kernel,
        grid=(_GRID,),
        in_specs=[
            pl.BlockSpec((_BB, _HIST), lambda i: (i, 0)),
            pl.BlockSpec((2, _DIM), lambda i: (0, 0)),
        ],
        out_specs=pl.BlockSpec(memory_space=pltpu.MemorySpace.HBM),
        out_shape=jax.ShapeDtypeStruct((_BATCH, _HIST, _DIM), table.dtype),
        scratch_shapes=[
            pltpu.VMEM((_NBUF, _BB, _HIST, _DIM), jnp.float32),
            pltpu.SemaphoreType.DMA((_NBUF,)),
        ],
        compiler_params=pltpu.CompilerParams(
            dimension_semantics=("arbitrary",),
        ),
    )(xi, rows01)
    return out
